# Initial kernel scaffold; baseline (speedup 1.0000x reference)
import functools

import jax
import jax.numpy as jnp
from jax import lax
from jax.experimental import pallas as pl
from jax.experimental.pallas import tpu as pltpu
from jax.experimental.pallas import tpu_sc as plsc

_N_VOX = 10000
_N_PTS = 320000
_D = 128
_B = 16
_SCALE = 14.285714285714286

_NC = 2
_NS = 16
_L = 16
_NW = _NC * _NS
_P = _N_PTS // _NW
_NV = _P // _L


def _sc_hist_body(v2p_hbm, bidx_hbm, out_hbm, v2p_v, b_v, hist_v, idx1_v,
                  hist_sh):
    cid = lax.axis_index("c")
    sid = lax.axis_index("s")
    wid = sid * _NC + cid
    base = wid * _P

    pltpu.sync_copy(v2p_hbm.at[pl.ds(base, _P)], v2p_v)
    pltpu.sync_copy(bidx_hbm.at[pl.ds(base, _P)], b_v)

    zeros = jnp.zeros((_L,), jnp.float32)

    def zero_hist(g, carry):
        hist_v[0, pl.ds(g * _L, _L)] = zeros
        return carry

    lax.fori_loop(0, _NV, zero_hist, 0)
    pltpu.sync_copy(hist_v, hist_sh.at[pl.ds(sid, 1)])
    plsc.subcore_barrier()

    b_lo = jnp.min(b_v[pl.ds(0, _L)])
    b_hi = jnp.max(b_v[pl.ds(_P - _L, _L)])

    def batch_pass(bb, carry):
        def accum(g, c):
            sl = pl.ds(g * _L, _L)
            v = v2p_v[sl]
            m = b_v[sl] == bb
            cnt, last = plsc.scan_count(v, mask=m)
            plsc.addupdate_scatter(
                hist_v.at[0], [v], cnt.astype(jnp.float32), mask=last)
            return c

        lax.fori_loop(0, _NV, accum, 0)
        idx1_v[0] = bb
        pltpu.sync_copy(hist_v, hist_sh.at[idx1_v], add=True)
        lax.fori_loop(0, _NV, zero_hist, 0)
        return carry

    lax.fori_loop(b_lo, b_hi + 1, batch_pass, 0)
    plsc.subcore_barrier()

    pltpu.sync_copy(hist_sh.at[pl.ds(sid, 1)], hist_v)
    pltpu.sync_copy(hist_v, out_hbm.at[cid, pl.ds(sid, 1)])


@functools.cache
def _sc_hist_kernel():
    mesh = plsc.VectorSubcoreMesh(
        core_axis_name="c", subcore_axis_name="s", num_cores=_NC,
        num_subcores=_NS)
    return pl.kernel(
        _sc_hist_body,
        out_type=jax.ShapeDtypeStruct((_NC, _NS, _N_VOX), jnp.float32),
        mesh=mesh,
        scratch_types=[
            pltpu.VMEM((_P,), jnp.int32),
            pltpu.VMEM((_P,), jnp.int32),
            pltpu.VMEM((1, _N_VOX), jnp.float32),
            pltpu.VMEM((1,), jnp.int32),
            pltpu.VMEM_SHARED((_B, _N_VOX), jnp.float32),
        ],
    )


def _tc_finish_body(hist_ref, feats_ref, cap_ref, out_ref):
    h = hist_ref[0] + hist_ref[1]
    counts = jnp.sum(h, axis=1, keepdims=True)
    sums = jnp.dot(h, feats_ref[...], preferred_element_type=jnp.float32)
    pooled = sums / jnp.maximum(counts, 1.0)
    sq = jnp.sum(pooled * pooled, axis=1, keepdims=True)
    pooled_n = pooled / jnp.maximum(jnp.sqrt(sq), 1e-12)
    logits = jnp.dot(pooled_n, cap_ref[...].T,
                     preferred_element_type=jnp.float32)
    out_ref[...] = logits * _SCALE


@functools.cache
def _tc_finish_kernel():
    return pl.pallas_call(
        _tc_finish_body,
        out_shape=jax.ShapeDtypeStruct((_B, _B), jnp.float32),
    )


def kernel(adapter_feats, caption_embed, v2p_map, batch_idx):
    v2p = v2p_map.astype(jnp.int32)
    bid = batch_idx.astype(jnp.int32)
    hist = _sc_hist_kernel()(v2p, bid)
    caption_logit = _tc_finish_kernel()(hist, adapter_feats, caption_embed)
    caption_labels = jnp.arange(_B, dtype=jnp.int64)
    return caption_logit, caption_labels


# speedup vs baseline: 58.3499x; 58.3499x over previous
import functools

import jax
import jax.numpy as jnp
from jax import lax
from jax.experimental import pallas as pl
from jax.experimental.pallas import tpu as pltpu
from jax.experimental.pallas import tpu_sc as plsc

_N_VOX = 10000
_N_PTS = 320000
_D = 128
_B = 16
_SCALE = 14.285714285714286

_NC = 2
_NS = 16
_L = 16
_NW = _NC * _NS
_P = _N_PTS // _NW
_NVP = 10112
_NVZ = _NVP // _L
_CW = 128
_NCH = _NVP // _CW
_NFULL = _P // _CW
_HSZ = _B * _NVP + _CW


def _sc_hist_body(v2p_hbm, bidx_hbm, out_hbm, v2p_v, b_v, idx_v, ones_v,
                  row_v, hist_sh, sem):
    cid = lax.axis_index("c")
    sid = lax.axis_index("s")
    wid = sid * _NC + cid
    base = wid * _P

    pltpu.sync_copy(v2p_hbm.at[pl.ds(base, _P)], v2p_v)
    pltpu.sync_copy(bidx_hbm.at[pl.ds(base, _P)], b_v)

    for c in range(_CW // _L):
        ones_v[pl.ds(c * _L, _L)] = jnp.ones((_L,), jnp.float32)
        idx_v[_NCH - 1, pl.ds(c * _L, _L)] = jnp.full(
            (_L,), _B * _NVP, jnp.int32)

    def zero_row(g, carry):
        row_v[pl.ds(g * _L, _L)] = jnp.zeros((_L,), jnp.float32)
        return carry

    lax.fori_loop(0, _NVZ, zero_row, 0)
    pltpu.sync_copy(row_v, hist_sh.at[pl.ds(sid * _NVP, _NVP)])

    def make_idx(j, carry):
        for c in range(_CW // _L):
            sl = pl.ds(j * _CW + c * _L, _L)
            flat = b_v[sl] * _NVP + v2p_v[sl]
            idx_v[j, pl.ds(c * _L, _L)] = flat
        return carry

    lax.fori_loop(0, _NFULL, make_idx, 0)
    tail = pl.ds(_NFULL * _CW, _L)
    idx_v[_NCH - 1, pl.ds(0, _L)] = b_v[tail] * _NVP + v2p_v[tail]

    plsc.subcore_barrier()

    copies = [
        pltpu.async_copy(
            ones_v, hist_sh.at[idx_v.at[j]], sem, add=True)
        for j in range(_NCH)
    ]
    for c in copies:
        c.wait()
    plsc.subcore_barrier()

    pltpu.sync_copy(hist_sh.at[pl.ds(sid * _NVP, _NVP)], row_v)
    pltpu.sync_copy(row_v, out_hbm.at[cid, sid])


@functools.cache
def _sc_hist_kernel():
    mesh = plsc.VectorSubcoreMesh(
        core_axis_name="c", subcore_axis_name="s", num_cores=_NC,
        num_subcores=_NS)
    return pl.kernel(
        _sc_hist_body,
        out_type=jax.ShapeDtypeStruct((_NC, _B, _NVP), jnp.float32),
        mesh=mesh,
        compiler_params=pltpu.CompilerParams(needs_layout_passes=False),
        scratch_types=[
            pltpu.VMEM((_P,), jnp.int32),
            pltpu.VMEM((_P,), jnp.int32),
            pltpu.VMEM((_NCH, _CW), jnp.int32),
            pltpu.VMEM((_CW,), jnp.float32),
            pltpu.VMEM((_NVP,), jnp.float32),
            pltpu.VMEM_SHARED((_HSZ,), jnp.float32),
            pltpu.SemaphoreType.DMA,
        ],
    )


def _tc_finish_body(hist_ref, feats_ref, cap_ref, out_ref):
    h = (hist_ref[0] + hist_ref[1])[:, :_N_VOX]
    counts = jnp.sum(h, axis=1, keepdims=True)
    sums = jnp.dot(h, feats_ref[...], preferred_element_type=jnp.float32)
    pooled = sums / jnp.maximum(counts, 1.0)
    sq = jnp.sum(pooled * pooled, axis=1, keepdims=True)
    pooled_n = pooled / jnp.maximum(jnp.sqrt(sq), 1e-12)
    logits = jnp.dot(pooled_n, cap_ref[...].T,
                     preferred_element_type=jnp.float32)
    out_ref[...] = logits * _SCALE


@functools.cache
def _tc_finish_kernel():
    return pl.pallas_call(
        _tc_finish_body,
        out_shape=jax.ShapeDtypeStruct((_B, _B), jnp.float32),
    )


def kernel(adapter_feats, caption_embed, v2p_map, batch_idx):
    v2p = v2p_map.astype(jnp.int32)
    bid = batch_idx.astype(jnp.int32)
    hist = _sc_hist_kernel()(v2p, bid)
    caption_logit = _tc_finish_kernel()(hist, adapter_feats, caption_embed)
    caption_labels = jnp.arange(_B, dtype=jnp.int64)
    return caption_logit, caption_labels
